# baseline (speedup 1.0000x reference)
import functools

import jax
import jax.numpy as jnp
from jax import lax
from jax.experimental import pallas as pl
from jax.experimental.pallas import tpu as pltpu
from jax.experimental.pallas import tpu_sc as plsc

DIM = 1024
HID = 4096
E = 8
NTOK = 4096
TB = 256
NH = 4
HC = HID // NH
NB = NTOK // TB + E
P = NB * TB

NC = 2
NS = 16
NW = NC * NS


def _make_row_gather(n_out, dwords, chunk):
    assert n_out % (8 * NW) == 0
    b_per_w = n_out // NW
    assert b_per_w % chunk == 0
    nchunks = b_per_w // chunk
    mesh = plsc.VectorSubcoreMesh(core_axis_name="c", subcore_axis_name="s")

    def body(table_hbm, idx_hbm, out_hbm, idx0, idx1, rows0, rows1, sem0,
             sem1):
        wid = lax.axis_index("s") * NC + lax.axis_index("c")
        base = wid * b_per_w
        idxs = (idx0, idx1)
        rows = (rows0, rows1)
        sems = (sem0, sem1)

        def start(ci):
            b = ci % 2
            pltpu.sync_copy(idx_hbm.at[pl.ds(base + ci * chunk, chunk)],
                            idxs[b])
            return pltpu.async_copy(table_hbm.at[idxs[b]], rows[b], sems[b])

        cps = [start(0)]
        for ci in range(nchunks):
            if ci + 1 < nchunks:
                cps.append(start(ci + 1))
            cps[ci].wait()
            pltpu.sync_copy(rows[ci % 2],
                            out_hbm.at[pl.ds(base + ci * chunk, chunk)])

    return pl.kernel(
        body,
        out_type=jax.ShapeDtypeStruct((n_out, dwords), jnp.int32),
        mesh=mesh,
        scratch_types=[
            pltpu.VMEM((chunk,), jnp.int32),
            pltpu.VMEM((chunk,), jnp.int32),
            pltpu.VMEM((chunk, dwords), jnp.int32),
            pltpu.VMEM((chunk, dwords), jnp.int32),
            pltpu.SemaphoreType.DMA,
            pltpu.SemaphoreType.DMA,
        ],
    )


def _make_row_scatter(n_src, n_out, dwords, chunk):
    assert n_src % (8 * NW) == 0
    b_per_w = n_src // NW
    assert b_per_w % chunk == 0
    nchunks = b_per_w // chunk
    mesh = plsc.VectorSubcoreMesh(core_axis_name="c", subcore_axis_name="s")

    def body(src_hbm, idx_hbm, out_hbm, idx0, idx1, rows0, rows1, sem0, sem1):
        wid = lax.axis_index("s") * NC + lax.axis_index("c")
        base = wid * b_per_w
        idxs = (idx0, idx1)
        rows = (rows0, rows1)
        sems = (sem0, sem1)

        def start(ci):
            b = ci % 2
            pltpu.sync_copy(idx_hbm.at[pl.ds(base + ci * chunk, chunk)],
                            idxs[b])
            pltpu.sync_copy(src_hbm.at[pl.ds(base + ci * chunk, chunk)],
                            rows[b])
            return pltpu.async_copy(rows[b], out_hbm.at[idxs[b]], sems[b])

        cps = [start(0)]
        for ci in range(nchunks):
            if ci + 1 < nchunks:
                cps.append(start(ci + 1))
            cps[ci].wait()

    return pl.kernel(
        body,
        out_type=jax.ShapeDtypeStruct((n_out, dwords), jnp.int32),
        mesh=mesh,
        scratch_types=[
            pltpu.VMEM((chunk,), jnp.int32),
            pltpu.VMEM((chunk,), jnp.int32),
            pltpu.VMEM((chunk, dwords), jnp.int32),
            pltpu.VMEM((chunk, dwords), jnp.int32),
            pltpu.SemaphoreType.DMA,
            pltpu.SemaphoreType.DMA,
        ],
    )


_scatter_in = _make_row_scatter(NTOK, P, DIM // 2, 64)
_gather_out = _make_row_gather(NTOK, DIM // 2, 64)


_R = NTOK // 128


def _route_body(ea_ref, dest_ref, binfo_ref):
    ea = ea_ref[...]
    lt128 = (lax.broadcasted_iota(jnp.int32, (128, 128), 0)
             <= lax.broadcasted_iota(jnp.int32, (128, 128), 1))
    lt128 = lt128.astype(jnp.float32)
    sr = (lax.broadcasted_iota(jnp.int32, (_R, _R), 0)
          > lax.broadcasted_iota(jnp.int32, (_R, _R), 1))
    sr = sr.astype(jnp.float32)

    rank = jnp.zeros((_R, 128), jnp.float32)
    base = jnp.zeros((_R, 128), jnp.float32)
    counts = []
    for e in range(E):
        m = (ea == e).astype(jnp.float32)
        lane_incl = jnp.dot(m, lt128, preferred_element_type=jnp.float32)
        row_tot = jnp.broadcast_to(lane_incl[:, 127:128], (_R, 128))
        row_excl = jnp.dot(sr, row_tot, preferred_element_type=jnp.float32)
        rank = rank + m * (lane_incl - m + row_excl)
        counts.append(row_excl[_R - 1, 0] + row_tot[_R - 1, 0])

    nbs = []
    pad_off = []
    tot = jnp.float32(0)
    for e in range(E):
        nb_e = jnp.floor((counts[e] + (TB - 1)) / TB)
        pad_off.append(tot * TB)
        nbs.append(nb_e)
        tot = tot + nb_e
    for e in range(E):
        m = (ea == e).astype(jnp.float32)
        base = base + m * pad_off[e]
    dest_ref[...] = (rank + base).astype(jnp.int32)

    for e in range(E):
        binfo_ref[0, e] = (pad_off[e] / TB).astype(jnp.int32)
        binfo_ref[1, e] = nbs[e].astype(jnp.int32)


def _route(ea2d):
    return pl.pallas_call(
        _route_body,
        in_specs=[pl.BlockSpec(memory_space=pltpu.VMEM)],
        out_specs=(pl.BlockSpec(memory_space=pltpu.VMEM),
                   pl.BlockSpec(memory_space=pltpu.SMEM)),
        out_shape=(jax.ShapeDtypeStruct((_R, 128), jnp.int32),
                   jax.ShapeDtypeStruct((2, E), jnp.int32)),
    )(ea2d)


MAXB = NTOK // TB


def _ffn_body(bref, xs_hbm, wg_ref, bg_ref, wi_ref, bi_ref, wo_ref, bo_ref,
              ys_hbm, xs_v, acc_ref, in_sem, out_sem):
    c = pl.program_id(0)
    e = c // NH
    h = c % NH

    @pl.when(c == 0)
    def _():
        cp = pltpu.make_async_copy(xs_hbm, xs_v, in_sem)
        cp.start()
        cp.wait()

    blk0 = bref[0, e]
    nb = bref[1, e]

    def _drain(n):
        def w(j, _):
            pltpu.make_async_copy(
                acc_ref.at[0], ys_hbm.at[pl.ds(0, TB)], out_sem).wait()
            return 0
        lax.fori_loop(0, n, w, 0)

    @pl.when(jnp.logical_and(h == 0, e > 0))
    def _():
        _drain(bref[1, jnp.maximum(e - 1, 0)])

    wg = wg_ref[0].astype(jnp.bfloat16)
    wi = wi_ref[0].astype(jnp.bfloat16)
    wo = wo_ref[0].astype(jnp.bfloat16)

    def blk(j, _):
        row0 = (blk0 + j) * TB
        x = xs_v[pl.ds(row0, TB), :]
        a = jnp.dot(x, wg, preferred_element_type=jnp.float32)
        a = a + bg_ref[0, 0]
        a = a * jax.nn.sigmoid(a)
        b = jnp.dot(x, wi, preferred_element_type=jnp.float32)
        b = b + bi_ref[0, 0]
        t = (a * b).astype(jnp.bfloat16)
        y = jnp.dot(t, wo, preferred_element_type=jnp.float32)
        prev = acc_ref[j].astype(jnp.float32)
        y = jnp.where(h == 0, y, prev + y)

        @pl.when(h == NH - 1)
        def _():
            acc_ref[j] = (y + bo_ref[0, 0]).astype(jnp.bfloat16)
            pltpu.make_async_copy(
                acc_ref.at[j], ys_hbm.at[pl.ds(row0, TB)], out_sem).start()

        @pl.when(h != NH - 1)
        def _():
            acc_ref[j] = y.astype(jnp.bfloat16)

        return 0

    lax.fori_loop(0, nb, blk, 0)

    @pl.when(c == E * NH - 1)
    def _():
        _drain(nb)


@functools.partial(jax.jit, static_argnums=())
def _ffn(binfo, xs, Wg, bg3, Wi, bi3, Wo, bo3):
    grid_spec = pltpu.PrefetchScalarGridSpec(
        num_scalar_prefetch=1,
        grid=(E * NH,),
        in_specs=[
            pl.BlockSpec(memory_space=pl.ANY),
            pl.BlockSpec((1, DIM, HC), lambda c, b: (c // NH, 0, c % NH)),
            pl.BlockSpec((1, 1, HC), lambda c, b: (c // NH, 0, c % NH)),
            pl.BlockSpec((1, DIM, HC), lambda c, b: (c // NH, 0, c % NH)),
            pl.BlockSpec((1, 1, HC), lambda c, b: (c // NH, 0, c % NH)),
            pl.BlockSpec((1, HC, DIM), lambda c, b: (c // NH, c % NH, 0)),
            pl.BlockSpec((1, 1, DIM), lambda c, b: (c // NH, 0, 0)),
        ],
        out_specs=pl.BlockSpec(memory_space=pl.ANY),
        scratch_shapes=[
            pltpu.VMEM((P, DIM), jnp.bfloat16),
            pltpu.VMEM((MAXB, TB, DIM), jnp.bfloat16),
            pltpu.SemaphoreType.DMA,
            pltpu.SemaphoreType.DMA,
        ],
    )
    return pl.pallas_call(
        _ffn_body,
        grid_spec=grid_spec,
        out_shape=jax.ShapeDtypeStruct((P, DIM), jnp.bfloat16),
    )(binfo, xs, Wg, bg3, Wi, bi3, Wo, bo3)


def kernel(x, expert_assign, Wg, bg, Wi, bi, Wo, bo):
    Bc, Tc, C = x.shape
    xf = x.reshape(-1, C)
    ea = expert_assign.astype(jnp.int32)

    dest2d, binfo = _route(ea.reshape(_R, 128))
    dest = dest2d.reshape(NTOK)

    x16 = xf.astype(jnp.bfloat16).reshape(NTOK, DIM // 2, 2)
    xw = lax.bitcast_convert_type(x16, jnp.int32)
    xsw = _scatter_in(xw, dest)
    xs = lax.bitcast_convert_type(xsw, jnp.bfloat16).reshape(P, DIM)
    ys = _ffn(binfo, xs, Wg, bg.reshape(E, 1, HID), Wi, bi.reshape(E, 1, HID),
              Wo, bo.reshape(E, 1, DIM))
    yw = lax.bitcast_convert_type(ys.reshape(P, DIM // 2, 2), jnp.int32)
    ow = _gather_out(yw, dest)
    o16 = lax.bitcast_convert_type(ow, jnp.bfloat16).reshape(NTOK, DIM)
    return o16.astype(jnp.float32).reshape(Bc, Tc, C)


# speedup vs baseline: 2.4655x; 2.4655x over previous
import functools

import jax
import jax.numpy as jnp
from jax import lax
from jax.experimental import pallas as pl
from jax.experimental.pallas import tpu as pltpu
from jax.experimental.pallas import tpu_sc as plsc

DIM = 1024
HID = 4096
E = 8
NTOK = 4096
TB = 256
NH = 4
HC = HID // NH
NB = NTOK // TB + E
P = NB * TB

NC = 2
NS = 16
NW = NC * NS


def _make_row_gather(n_out, chunk):
    assert n_out % (8 * NW) == 0
    b_per_w = n_out // NW
    assert b_per_w % chunk == 0
    nchunks = b_per_w // chunk
    mesh = plsc.VectorSubcoreMesh(core_axis_name="c", subcore_axis_name="s")

    def body(table_hbm, idx_hbm, out_hbm, idx0, idx1, rows0, rows1, sem0,
             sem1):
        wid = lax.axis_index("s") * NC + lax.axis_index("c")
        base = wid * b_per_w
        idxs = (idx0, idx1)
        rows = (rows0, rows1)
        sems = (sem0, sem1)

        def start(ci):
            b = ci % 2
            pltpu.sync_copy(idx_hbm.at[pl.ds(base + ci * chunk, chunk)],
                            idxs[b])
            return pltpu.async_copy(table_hbm.at[idxs[b]], rows[b], sems[b])

        cps = [start(0)]
        for ci in range(nchunks):
            if ci + 1 < nchunks:
                cps.append(start(ci + 1))
            cps[ci].wait()
            pltpu.sync_copy(rows[ci % 2],
                            out_hbm.at[pl.ds(base + ci * chunk, chunk)])

    return pl.kernel(
        body,
        out_type=jax.ShapeDtypeStruct((n_out, DIM), jnp.float32),
        mesh=mesh,
        scratch_types=[
            pltpu.VMEM((chunk,), jnp.int32),
            pltpu.VMEM((chunk,), jnp.int32),
            pltpu.VMEM((chunk, DIM), jnp.float32),
            pltpu.VMEM((chunk, DIM), jnp.float32),
            pltpu.SemaphoreType.DMA,
            pltpu.SemaphoreType.DMA,
        ],
    )


def _make_row_scatter(n_src, n_out, chunk):
    assert n_src % (8 * NW) == 0
    b_per_w = n_src // NW
    assert b_per_w % chunk == 0
    nchunks = b_per_w // chunk
    mesh = plsc.VectorSubcoreMesh(core_axis_name="c", subcore_axis_name="s")

    def body(src_hbm, idx_hbm, out_hbm, idx0, idx1, rows0, rows1, sem0, sem1):
        wid = lax.axis_index("s") * NC + lax.axis_index("c")
        base = wid * b_per_w
        idxs = (idx0, idx1)
        rows = (rows0, rows1)
        sems = (sem0, sem1)

        def start(ci):
            b = ci % 2
            pltpu.sync_copy(idx_hbm.at[pl.ds(base + ci * chunk, chunk)],
                            idxs[b])
            pltpu.sync_copy(src_hbm.at[pl.ds(base + ci * chunk, chunk)],
                            rows[b])
            return pltpu.async_copy(rows[b], out_hbm.at[idxs[b]], sems[b])

        cps = [start(0)]
        for ci in range(nchunks):
            if ci + 1 < nchunks:
                cps.append(start(ci + 1))
            cps[ci].wait()

    return pl.kernel(
        body,
        out_type=jax.ShapeDtypeStruct((n_out, DIM), jnp.float32),
        mesh=mesh,
        scratch_types=[
            pltpu.VMEM((chunk,), jnp.int32),
            pltpu.VMEM((chunk,), jnp.int32),
            pltpu.VMEM((chunk, DIM), jnp.float32),
            pltpu.VMEM((chunk, DIM), jnp.float32),
            pltpu.SemaphoreType.DMA,
            pltpu.SemaphoreType.DMA,
        ],
    )


_scatter_in = _make_row_scatter(NTOK, P, 32)
_gather_out = _make_row_gather(NTOK, 32)


_R = NTOK // 128


def _route_body(ea_ref, dest_ref, sinfo_ref):
    ea = ea_ref[...]
    lt128 = (lax.broadcasted_iota(jnp.int32, (128, 128), 0)
             <= lax.broadcasted_iota(jnp.int32, (128, 128), 1))
    lt128 = lt128.astype(jnp.float32)
    sr = (lax.broadcasted_iota(jnp.int32, (_R, _R), 0)
          > lax.broadcasted_iota(jnp.int32, (_R, _R), 1))
    sr = sr.astype(jnp.float32)

    rank = jnp.zeros((_R, 128), jnp.float32)
    base = jnp.zeros((_R, 128), jnp.float32)
    counts = []
    for e in range(E):
        m = (ea == e).astype(jnp.float32)
        lane_incl = jnp.dot(m, lt128, preferred_element_type=jnp.float32)
        row_tot = jnp.broadcast_to(lane_incl[:, 127:128], (_R, 128))
        row_excl = jnp.dot(sr, row_tot, preferred_element_type=jnp.float32)
        rank = rank + m * (lane_incl - m + row_excl)
        counts.append(row_excl[_R - 1, 0] + row_tot[_R - 1, 0])

    cb = []
    pad_off = []
    tot = jnp.float32(0)
    for e in range(E):
        nb_e = jnp.floor((counts[e] + (TB - 1)) / TB)
        pad_off.append(tot * TB)
        tot = tot + nb_e
        cb.append(tot)
    for e in range(E):
        m = (ea == e).astype(jnp.float32)
        base = base + m * pad_off[e]
    dest_ref[...] = (rank + base).astype(jnp.int32)

    g = lax.broadcasted_iota(jnp.int32, (1, 128), 1).astype(jnp.float32)
    raw = jnp.zeros((1, 128), jnp.float32)
    laste = jnp.float32(0)
    nbs = [cb[e] - (cb[e - 1] if e else jnp.float32(0)) for e in range(E)]
    for e in range(E):
        raw = raw + (g >= cb[e]).astype(jnp.float32)
        laste = jnp.where(nbs[e] > 0, jnp.float32(e), laste)
    emap = jnp.minimum(raw, laste)
    valid = (g < cb[E - 1]).astype(jnp.float32)

    firste = jnp.float32(E - 1)
    for e in range(E - 1, -1, -1):
        firste = jnp.where(nbs[e] > 0, jnp.float32(e), firste)
    nextexp = []
    for e in range(E):
        cand = firste
        for e2 in range(E - 1, e, -1):
            cand = jnp.where(nbs[e2] > 0, jnp.float32(e2), cand)
        nextexp.append(cand)
    first = jnp.zeros((1, 128), jnp.float32)
    nxe = jnp.zeros((1, 128), jnp.float32)
    nruns = jnp.float32(0)
    for e in range(E):
        blk0 = cb[e] - nbs[e]
        has = (nbs[e] > 0).astype(jnp.float32)
        first = first + has * (g == blk0).astype(jnp.float32)
        nxe = nxe + (emap == e).astype(jnp.float32) * nextexp[e]
        nruns = nruns + has
    runid = jnp.dot(first, lt128, preferred_element_type=jnp.float32) - 1.0
    hincr = (emap == laste).astype(jnp.float32)

    sinfo_ref[0:1, :] = emap.astype(jnp.int32)
    sinfo_ref[1:2, :] = valid.astype(jnp.int32)
    sinfo_ref[2:3, :] = first.astype(jnp.int32)
    sinfo_ref[3:4, :] = runid.astype(jnp.int32)
    sinfo_ref[4:5, :] = nxe.astype(jnp.int32)
    sinfo_ref[5:6, :] = hincr.astype(jnp.int32)
    sinfo_ref[6:7, :] = jnp.broadcast_to(nruns, (1, 128)).astype(jnp.int32)
    sinfo_ref[7:8, :] = jnp.zeros((1, 128), jnp.int32)


def _route(ea2d):
    return pl.pallas_call(
        _route_body,
        out_shape=(jax.ShapeDtypeStruct((_R, 128), jnp.int32),
                   jax.ShapeDtypeStruct((8, 128), jnp.int32)),
    )(ea2d)


def _ffn_body(sref, x_ref, wg_hbm, bg_ref, wi_hbm, bi_ref, wo_hbm, bo_ref,
              out_ref, acc_ref, wgb, wib, wob, wsem):
    h = pl.program_id(0)
    g = pl.program_id(1)

    def fetch(e, hh, slot):
        pltpu.make_async_copy(
            wg_hbm.at[e, :, pl.ds(hh * HC, HC)], wgb.at[slot], wsem).start()
        pltpu.make_async_copy(
            wi_hbm.at[e, :, pl.ds(hh * HC, HC)], wib.at[slot], wsem).start()
        pltpu.make_async_copy(
            wo_hbm.at[e, pl.ds(hh * HC, HC), :], wob.at[slot], wsem).start()

    def drain():
        pltpu.make_async_copy(
            wg_hbm.at[0, :, pl.ds(0, HC)], wgb.at[0], wsem).wait()
        pltpu.make_async_copy(
            wi_hbm.at[0, :, pl.ds(0, HC)], wib.at[0], wsem).wait()
        pltpu.make_async_copy(
            wo_hbm.at[0, pl.ds(0, HC), :], wob.at[0], wsem).wait()

    nrun = sref[6, 0]
    par = (h * nrun + sref[3, g]) % 2

    @pl.when(jnp.logical_and(h == 0, g == 0))
    def _():
        fetch(sref[0, 0], 0, 0)

    @pl.when(sref[2, g] == 1)
    def _():
        drain()
        ne = sref[4, g]
        nh = jnp.minimum(h + sref[5, g], NH - 1)
        fetch(ne, nh, 1 - par)

    @pl.when(sref[1, g] == 1)
    def _():
        x = x_ref[...].astype(jnp.bfloat16)
        wg = wgb[par].astype(jnp.bfloat16)
        wi = wib[par].astype(jnp.bfloat16)
        wo = wob[par].astype(jnp.bfloat16)
        a = jnp.dot(x, wg, preferred_element_type=jnp.float32)
        a = a + bg_ref[0, 0]
        a = a * jax.nn.sigmoid(a)
        b = jnp.dot(x, wi, preferred_element_type=jnp.float32)
        b = b + bi_ref[0, 0]
        t = (a * b).astype(jnp.bfloat16)
        y = jnp.dot(t, wo, preferred_element_type=jnp.float32)

        @pl.when(h == 0)
        def _():
            acc_ref[g] = y

        @pl.when(jnp.logical_and(h != 0, h != NH - 1))
        def _():
            acc_ref[g] = acc_ref[g] + y

        @pl.when(h == NH - 1)
        def _():
            out_ref[...] = acc_ref[g] + y + bo_ref[0, 0]

    @pl.when(jnp.logical_and(h == NH - 1, g == NB - 1))
    def _():
        drain()


@functools.partial(jax.jit, static_argnums=())
def _ffn(sinfo, xs, Wg, bg3, Wi, bi3, Wo, bo3):
    grid_spec = pltpu.PrefetchScalarGridSpec(
        num_scalar_prefetch=1,
        grid=(NH, NB),
        in_specs=[
            pl.BlockSpec((TB, DIM), lambda h, g, s: (g, 0)),
            pl.BlockSpec(memory_space=pl.ANY),
            pl.BlockSpec((1, 1, HC), lambda h, g, s: (s[0, g], 0, h)),
            pl.BlockSpec(memory_space=pl.ANY),
            pl.BlockSpec((1, 1, HC), lambda h, g, s: (s[0, g], 0, h)),
            pl.BlockSpec(memory_space=pl.ANY),
            pl.BlockSpec((1, 1, DIM), lambda h, g, s: (s[0, g], 0, 0)),
        ],
        out_specs=pl.BlockSpec((TB, DIM), lambda h, g, s: (g, 0)),
        scratch_shapes=[
            pltpu.VMEM((NB, TB, DIM), jnp.float32),
            pltpu.VMEM((2, DIM, HC), jnp.float32),
            pltpu.VMEM((2, DIM, HC), jnp.float32),
            pltpu.VMEM((2, HC, DIM), jnp.float32),
            pltpu.SemaphoreType.DMA,
        ],
    )
    return pl.pallas_call(
        _ffn_body,
        grid_spec=grid_spec,
        out_shape=jax.ShapeDtypeStruct((P, DIM), jnp.float32),
    )(sinfo, xs, Wg, bg3, Wi, bi3, Wo, bo3)


def kernel(x, expert_assign, Wg, bg, Wi, bi, Wo, bo):
    Bc, Tc, C = x.shape
    xf = x.reshape(-1, C)
    ea = expert_assign.astype(jnp.int32)

    dest2d, sinfo = _route(ea.reshape(_R, 128))
    dest = dest2d.reshape(NTOK)

    xs = _scatter_in(xf, dest)
    ys = _ffn(sinfo, xs, Wg, bg.reshape(E, 1, HID), Wi, bi.reshape(E, 1, HID),
              Wo, bo.reshape(E, 1, DIM))
    of = _gather_out(ys, dest)
    return of.reshape(Bc, Tc, C)
